# baseline (speedup 1.0000x reference)
import jax
import jax.numpy as jnp
from jax.experimental import pallas as pl


def _matvec_body(d_ref, w_ref, b_ref, o_ref):
    d = d_ref[0]
    w = w_ref[0]
    acc = jnp.sum(d * w, axis=1)
    o_ref[0] = acc[None, :] + b_ref[0, 0, 0]


def kernel(dict, coef):
    D = dict
    B, L, K = D.shape
    w = coef[:, :K].reshape(B, 1, K)
    bias = jnp.broadcast_to(coef[:, K:].reshape(B, 1, 1), (B, 1, 128))

    out = pl.pallas_call(
        _matvec_body,
        grid=(B,),
        in_specs=[
            pl.BlockSpec((1, L, K), lambda b: (b, 0, 0)),
            pl.BlockSpec((1, 1, K), lambda b: (b, 0, 0)),
            pl.BlockSpec((1, 1, 128), lambda b: (b, 0, 0)),
        ],
        out_specs=pl.BlockSpec((1, 1, L), lambda b: (b, 0, 0)),
        out_shape=jax.ShapeDtypeStruct((B, 1, L), jnp.float32),
    )(D, w, bias)
    return out.reshape(B, L, 1)


# speedup vs baseline: 1.1904x; 1.1904x over previous
import jax
import jax.numpy as jnp
from jax.experimental import pallas as pl


_BB = 2


def _matvec_body(d_ref, w_ref, b_ref, o_ref):
    for i in range(_BB):
        d = d_ref[i]
        w = w_ref[i]
        acc = jnp.sum(d * w, axis=1)
        o_ref[i] = acc[None, :] + b_ref[i, 0, 0]


def kernel(dict, coef):
    D = dict
    B, L, K = D.shape
    w = coef[:, :K].reshape(B, 1, K)
    bias = jnp.broadcast_to(coef[:, K:].reshape(B, 1, 1), (B, 1, 128))

    out = pl.pallas_call(
        _matvec_body,
        grid=(B // _BB,),
        in_specs=[
            pl.BlockSpec((_BB, L, K), lambda b: (b, 0, 0)),
            pl.BlockSpec((_BB, 1, K), lambda b: (b, 0, 0)),
            pl.BlockSpec((_BB, 1, 128), lambda b: (b, 0, 0)),
        ],
        out_specs=pl.BlockSpec((_BB, 1, L), lambda b: (b, 0, 0)),
        out_shape=jax.ShapeDtypeStruct((B, 1, L), jnp.float32),
    )(D, w, bias)
    return out.reshape(B, L, 1)


# speedup vs baseline: 1.2128x; 1.0187x over previous
import jax
import jax.numpy as jnp
from jax.experimental import pallas as pl


_BB = 4


def _matvec_body(d_ref, w_ref, b_ref, o_ref):
    for i in range(_BB):
        d = d_ref[i]
        w = w_ref[i]
        acc = jnp.sum(d * w, axis=1)
        o_ref[i] = acc[None, :] + b_ref[i, 0, 0]


def kernel(dict, coef):
    D = dict
    B, L, K = D.shape
    w = coef[:, :K].reshape(B, 1, K)
    bias = jnp.broadcast_to(coef[:, K:].reshape(B, 1, 1), (B, 1, 128))

    out = pl.pallas_call(
        _matvec_body,
        grid=(B // _BB,),
        in_specs=[
            pl.BlockSpec((_BB, L, K), lambda b: (b, 0, 0)),
            pl.BlockSpec((_BB, 1, K), lambda b: (b, 0, 0)),
            pl.BlockSpec((_BB, 1, 128), lambda b: (b, 0, 0)),
        ],
        out_specs=pl.BlockSpec((_BB, 1, L), lambda b: (b, 0, 0)),
        out_shape=jax.ShapeDtypeStruct((B, 1, L), jnp.float32),
    )(D, w, bias)
    return out.reshape(B, L, 1)


# speedup vs baseline: 1.2206x; 1.0064x over previous
import jax
import jax.numpy as jnp
from jax.experimental import pallas as pl


_BB = 4
_HB = _BB // 2


def _matvec_body(d0_ref, d1_ref, w_ref, b_ref, o_ref):
    for j, d_ref in enumerate((d0_ref, d1_ref)):
        for i in range(_HB):
            d = d_ref[i]
            w = w_ref[j * _HB + i]
            acc = jnp.sum(d * w, axis=1)
            o_ref[j * _HB + i] = acc[None, :] + b_ref[j * _HB + i, 0, 0]


def kernel(dict, coef):
    D = dict
    B, L, K = D.shape
    w = coef[:, :K].reshape(B, 1, K)
    bias = jnp.broadcast_to(coef[:, K:].reshape(B, 1, 1), (B, 1, 128))

    out = pl.pallas_call(
        _matvec_body,
        grid=(B // _BB,),
        in_specs=[
            pl.BlockSpec((_HB, L, K), lambda b: (2 * b, 0, 0)),
            pl.BlockSpec((_HB, L, K), lambda b: (2 * b + 1, 0, 0)),
            pl.BlockSpec((_BB, 1, K), lambda b: (b, 0, 0)),
            pl.BlockSpec((_BB, 1, 128), lambda b: (b, 0, 0)),
        ],
        out_specs=pl.BlockSpec((_BB, 1, L), lambda b: (b, 0, 0)),
        out_shape=jax.ShapeDtypeStruct((B, 1, L), jnp.float32),
    )(D, D, w, bias)
    return out.reshape(B, L, 1)
